# baseline (speedup 1.0000x reference)
import jax
import jax.numpy as jnp
from jax.experimental import pallas as pl

_DEPTH = 1000
_ROWS = 1024


def _onehot_block(idx_ref, out_ref):
    idx = idx_ref[...]
    iota = jax.lax.broadcasted_iota(jnp.int32, out_ref.shape, 1)
    out_ref[...] = jnp.where(idx == iota, jnp.float32(1.0), jnp.float32(0.0))


def kernel(indices):
    B, P = indices.shape
    n = B * P
    idx2 = indices.astype(jnp.int32).reshape(n, 1)
    out2 = pl.pallas_call(
        _onehot_block,
        grid=(n // _ROWS,),
        in_specs=[pl.BlockSpec((_ROWS, 1), lambda i: (i, 0))],
        out_specs=pl.BlockSpec((_ROWS, _DEPTH), lambda i: (i, 0)),
        out_shape=jax.ShapeDtypeStruct((n, _DEPTH), jnp.float32),
    )(idx2)
    return out2.reshape(B, P, _DEPTH)
